# Initial kernel scaffold; baseline (speedup 1.0000x reference)
import jax
import jax.numpy as jnp
from jax.experimental import pallas as pl


def kernel(x, edge_index, batch, edge_weights, W1, b1, W2, b2, W3, b3, Wl, bl):
    raise NotImplementedError("write your pallas kernel here")


# speedup vs baseline: 17.2702x; 17.2702x over previous
import functools

import jax
import jax.numpy as jnp
from jax import lax
from jax.experimental import pallas as pl
from jax.experimental.pallas import tpu as pltpu
from jax.experimental.pallas import tpu_sc as plsc

N = 10000
F = 128
H = 20
G = 64
C = 2

NC = 2
NS = 16
NW = NC * NS
HP = 32
CB = 128
CH = 80
EPW = CH * CB
EPAD = NW * EPW
NPAD = 10240
RPW = NPAD // NS
GSEG = 65

_MESH = plsc.VectorSubcoreMesh(core_axis_name="c", subcore_axis_name="s")
_SC_PARAMS = pltpu.CompilerParams(needs_layout_passes=False,
                                  use_tc_tiling_on_sc=False)


def _wid():
    return lax.axis_index("s") * NC + lax.axis_index("c")


@functools.partial(
    pl.kernel,
    out_type=jax.ShapeDtypeStruct((NC, NPAD), jnp.float32),
    mesh=_MESH,
    compiler_params=_SC_PARAMS,
    scratch_types=[
        pltpu.VMEM((CH, CB), jnp.int32),
        pltpu.VMEM((CH, CB), jnp.float32),
        pltpu.VMEM((RPW,), jnp.float32),
        pltpu.VMEM_SHARED((NPAD,), jnp.float32),
    ],
)
def _deg_kernel(dst_hbm, ew_hbm, deg_out, dst_v, ew_v, stage_v, deg_s):
    s = lax.axis_index("s")
    c = lax.axis_index("c")
    w = _wid()
    pltpu.sync_copy(dst_hbm.at[w], dst_v)
    pltpu.sync_copy(ew_hbm.at[w], ew_v)

    def zero(i, carry):
        stage_v[pl.ds(i * 16, 16)] = jnp.zeros((16,), jnp.float32)
        return carry

    lax.fori_loop(0, RPW // 16, zero, 0)
    pltpu.sync_copy(stage_v, deg_s.at[pl.ds(s * RPW, RPW)])
    plsc.subcore_barrier()

    def body(j, carry):
        pltpu.sync_copy(ew_v.at[j], deg_s.at[dst_v.at[j]], add=True)
        return carry

    lax.fori_loop(0, CH, body, 0)
    plsc.subcore_barrier()
    pltpu.sync_copy(deg_s.at[pl.ds(s * RPW, RPW)], stage_v)
    pltpu.sync_copy(stage_v, deg_out.at[c].at[pl.ds(s * RPW, RPW)])


@functools.partial(
    pl.kernel,
    out_type=jax.ShapeDtypeStruct((NW, CH, CB), jnp.float32),
    mesh=_MESH,
    compiler_params=_SC_PARAMS,
    scratch_types=[
        pltpu.VMEM((NPAD,), jnp.float32),
        pltpu.VMEM((CH, CB), jnp.int32),
        pltpu.VMEM((CH, CB), jnp.int32),
        pltpu.VMEM((CH, CB), jnp.float32),
        pltpu.VMEM((CH, CB), jnp.float32),
    ],
)
def _norm_kernel(dinv_hbm, src_hbm, dst_hbm, ew_hbm, norm_out,
                 dinv_v, src_v, dst_v, ew_v, norm_v):
    w = _wid()
    pltpu.sync_copy(dinv_hbm, dinv_v)
    pltpu.sync_copy(src_hbm.at[w], src_v)
    pltpu.sync_copy(dst_hbm.at[w], dst_v)
    pltpu.sync_copy(ew_hbm.at[w], ew_v)

    def body(j, carry):
        for r in range(0, CB, 16):
            sv = src_v[j, pl.ds(r, 16)]
            dv = dst_v[j, pl.ds(r, 16)]
            ev = ew_v[j, pl.ds(r, 16)]
            da = plsc.load_gather(dinv_v, [sv])
            db = plsc.load_gather(dinv_v, [dv])
            norm_v[j, pl.ds(r, 16)] = da * ev * db
        return carry

    lax.fori_loop(0, CH, body, 0)
    pltpu.sync_copy(norm_v, norm_out.at[w])


@functools.partial(
    pl.kernel,
    out_type=jax.ShapeDtypeStruct((NC, NPAD, HP), jnp.float32),
    mesh=_MESH,
    compiler_params=_SC_PARAMS,
    scratch_types=[
        pltpu.VMEM((CH, CB), jnp.int32),
        pltpu.VMEM((CH, CB), jnp.int32),
        pltpu.VMEM((CH, CB), jnp.float32),
        pltpu.VMEM((CB, HP), jnp.float32),
        pltpu.VMEM((RPW, HP), jnp.float32),
        pltpu.VMEM_SHARED((NPAD, HP), jnp.float32),
        pltpu.SemaphoreType.DMA,
    ],
)
def _agg_kernel(xw_hbm, src_hbm, dst_hbm, nrm_hbm, acc_out,
                src_v, dst_v, nrm_v, rows_v, stage_v, acc_s, sem):
    s = lax.axis_index("s")
    c = lax.axis_index("c")
    w = _wid()
    pltpu.sync_copy(src_hbm.at[w], src_v)
    pltpu.sync_copy(dst_hbm.at[w], dst_v)
    pltpu.sync_copy(nrm_hbm.at[w], nrm_v)

    def zero(i, carry):
        stage_v[i, pl.ds(0, 16)] = jnp.zeros((16,), jnp.float32)
        stage_v[i, pl.ds(16, 16)] = jnp.zeros((16,), jnp.float32)
        return carry

    lax.fori_loop(0, RPW, zero, 0)
    pltpu.sync_copy(stage_v, acc_s.at[pl.ds(s * RPW, RPW)])
    plsc.subcore_barrier()

    def body(j, carry):
        pltpu.async_copy(xw_hbm.at[src_v.at[j]], rows_v, sem).wait()
        for rr in range(CB // 16):
            nv = nrm_v[j, pl.ds(rr * 16, 16)]
            for q in range(16):
                r = rr * 16 + q
                sc = nv[q]
                rows_v[r, pl.ds(0, 16)] = rows_v[r, pl.ds(0, 16)] * sc
                rows_v[r, pl.ds(16, 16)] = rows_v[r, pl.ds(16, 16)] * sc
        pltpu.sync_copy(rows_v, acc_s.at[dst_v.at[j]], add=True)
        return carry

    lax.fori_loop(0, CH, body, 0)
    plsc.subcore_barrier()
    pltpu.sync_copy(acc_s.at[pl.ds(s * RPW, RPW)], stage_v)
    pltpu.sync_copy(stage_v, acc_out.at[c].at[pl.ds(s * RPW, RPW)])


@functools.partial(
    pl.kernel,
    out_type=(
        jax.ShapeDtypeStruct((NW, GSEG, HP), jnp.float32),
        jax.ShapeDtypeStruct((NW, GSEG, HP), jnp.float32),
    ),
    mesh=_MESH,
    compiler_params=_SC_PARAMS,
    scratch_types=[
        pltpu.VMEM((NPAD // NW, HP), jnp.float32),
        pltpu.VMEM((NPAD // NW + 16,), jnp.int32),
        pltpu.VMEM((GSEG, HP), jnp.float32),
        pltpu.VMEM((GSEG, HP), jnp.float32),
    ],
)
def _pool_kernel(h_hbm, batch_hbm, pmax_out, psum_out, h_v, b_v, pmax_v, psum_v):
    w = _wid()
    rows = NPAD // NW
    pltpu.sync_copy(h_hbm.at[pl.ds(w * rows, rows)], h_v)
    pltpu.sync_copy(batch_hbm.at[w], b_v.at[pl.ds(0, rows)])

    neg = jnp.full((16,), -jnp.inf, jnp.float32)
    zer = jnp.zeros((16,), jnp.float32)

    def init(g, carry):
        pmax_v[g, pl.ds(0, 16)] = neg
        pmax_v[g, pl.ds(16, 16)] = neg
        psum_v[g, pl.ds(0, 16)] = zer
        psum_v[g, pl.ds(16, 16)] = zer
        return carry

    lax.fori_loop(0, GSEG, init, 0)

    def body(i, carry):
        g = b_v[pl.ds(i, 16)][0]
        h0 = h_v[i, pl.ds(0, 16)]
        h1 = h_v[i, pl.ds(16, 16)]
        pmax_v[g, pl.ds(0, 16)] = jnp.maximum(pmax_v[g, pl.ds(0, 16)], h0)
        pmax_v[g, pl.ds(16, 16)] = jnp.maximum(pmax_v[g, pl.ds(16, 16)], h1)
        psum_v[g, pl.ds(0, 16)] = psum_v[g, pl.ds(0, 16)] + h0
        psum_v[g, pl.ds(16, 16)] = psum_v[g, pl.ds(16, 16)] + h1
        return carry

    lax.fori_loop(0, rows, body, 0)
    pltpu.sync_copy(pmax_v, pmax_out.at[w])
    pltpu.sync_copy(psum_v, psum_out.at[w])


def _tc_prep_body(deg_ref, x_ref, w_ref, dinv_ref, xw_ref):
    d = deg_ref[0] + deg_ref[1] + 1.0
    dinv_ref[...] = jnp.where(d > 0, lax.rsqrt(d), 0.0)
    xw_ref[...] = jnp.dot(x_ref[...], w_ref[...],
                          preferred_element_type=jnp.float32)


def _tc_prep(deg2, xp, w1p):
    blk = 1024
    grid = NPAD // blk
    return pl.pallas_call(
        _tc_prep_body,
        grid=(grid,),
        in_specs=[
            pl.BlockSpec((NC, blk, 1), lambda i: (0, i, 0)),
            pl.BlockSpec((blk, F), lambda i: (i, 0)),
            pl.BlockSpec((F, HP), lambda i: (0, 0)),
        ],
        out_specs=[
            pl.BlockSpec((blk, 1), lambda i: (i, 0)),
            pl.BlockSpec((blk, HP), lambda i: (i, 0)),
        ],
        out_shape=[
            jax.ShapeDtypeStruct((NPAD, 1), jnp.float32),
            jax.ShapeDtypeStruct((NPAD, HP), jnp.float32),
        ],
    )(deg2, xp, w1p)


def _tc_layer_body(acc_ref, xw_ref, dinv_ref, b_ref, wn_ref, *out_refs,
                   last):
    h_ref = out_refs[0]
    dinv = dinv_ref[...]
    t = acc_ref[0] + acc_ref[1] + dinv * dinv * xw_ref[...] + b_ref[...]
    nrm = jnp.sqrt(jnp.sum(t * t, axis=1, keepdims=True))
    h = jnp.maximum(t / jnp.maximum(nrm, 1e-12), 0.0)
    if last:
        lane = lax.broadcasted_iota(jnp.int32, h.shape, 1)
        h = jnp.where(lane == HP - 1, 1.0, h)
        h_ref[...] = h
    else:
        h_ref[...] = h
        out_refs[1][...] = jnp.dot(h, wn_ref[...],
                                   preferred_element_type=jnp.float32)


def _tc_layer(acc2, xw, dinv, b, wn, last=False):
    blk = 1024
    grid = NPAD // blk
    out_shape = [jax.ShapeDtypeStruct((NPAD, HP), jnp.float32)]
    out_specs = [pl.BlockSpec((blk, HP), lambda i: (i, 0))]
    if not last:
        out_shape.append(jax.ShapeDtypeStruct((NPAD, HP), jnp.float32))
        out_specs.append(pl.BlockSpec((blk, HP), lambda i: (i, 0)))
    res = pl.pallas_call(
        functools.partial(_tc_layer_body, last=last),
        grid=(grid,),
        in_specs=[
            pl.BlockSpec((NC, blk, HP), lambda i: (0, i, 0)),
            pl.BlockSpec((blk, HP), lambda i: (i, 0)),
            pl.BlockSpec((blk, 1), lambda i: (i, 0)),
            pl.BlockSpec((1, HP), lambda i: (0, 0)),
            pl.BlockSpec((HP, HP), lambda i: (0, 0)),
        ],
        out_specs=out_specs,
        out_shape=out_shape,
    )(acc2, xw, dinv, b, wn)
    return res if not last else (res[0], None)


def _tc_head_body(pmax_ref, psum_ref, wmax_ref, wmean_ref, bl_ref, out_ref):
    om = jnp.max(pmax_ref[...], axis=0)
    ps = jnp.sum(psum_ref[...], axis=0)
    cnt = ps[:, HP - 1:HP]
    mean = ps / jnp.maximum(cnt, 1.0)
    res = (jnp.dot(om, wmax_ref[...], preferred_element_type=jnp.float32)
           + jnp.dot(mean, wmean_ref[...], preferred_element_type=jnp.float32)
           + bl_ref[...])
    out_ref[...] = res[:G]


def _tc_head(pmax, psum, wmax, wmean, bl2):
    return pl.pallas_call(
        _tc_head_body,
        out_shape=jax.ShapeDtypeStruct((G, C), jnp.float32),
    )(pmax, psum, wmax, wmean, bl2)


def kernel(x, edge_index, batch, edge_weights, W1, b1, W2, b2, W3, b3, Wl, bl):
    f32 = jnp.float32
    src = edge_index[0].astype(jnp.int32)
    dst = edge_index[1].astype(jnp.int32)

    pad_e = EPAD - src.shape[0]
    padi = jnp.full((pad_e,), N, jnp.int32)
    srcp = jnp.concatenate([src, padi]).reshape(NW, CH, CB)
    dstp = jnp.concatenate([dst, padi]).reshape(NW, CH, CB)
    ewp = jnp.concatenate([edge_weights.astype(f32),
                           jnp.zeros((pad_e,), f32)]).reshape(NW, CH, CB)

    xp = jnp.zeros((NPAD, F), f32).at[:N].set(x.astype(f32))
    w1p = jnp.zeros((F, HP), f32).at[:, :H].set(W1.astype(f32))
    w2p = jnp.zeros((HP, HP), f32).at[:H, :H].set(W2.astype(f32))
    w3p = jnp.zeros((HP, HP), f32).at[:H, :H].set(W3.astype(f32))
    b1p = jnp.zeros((1, HP), f32).at[0, :H].set(b1.astype(f32))
    b2p = jnp.zeros((1, HP), f32).at[0, :H].set(b2.astype(f32))
    b3p = jnp.zeros((1, HP), f32).at[0, :H].set(b3.astype(f32))
    wmax = jnp.zeros((HP, C), f32).at[:H].set(Wl[:H].astype(f32))
    wmean = jnp.zeros((HP, C), f32).at[:H].set(Wl[H:2 * H].astype(f32))
    bl2 = bl.astype(f32).reshape(1, C)
    batchp = jnp.concatenate(
        [batch.astype(jnp.int32), jnp.full((NPAD - N,), G, jnp.int32)]
    ).reshape(NW, NPAD // NW)

    deg2 = _deg_kernel(dstp, ewp)
    deg3 = deg2.reshape(NC, NPAD, 1)
    dinv, xw1 = _tc_prep(deg3, xp, w1p)
    nrm = _norm_kernel(dinv.reshape(NPAD), srcp, dstp, ewp)

    acc1 = _agg_kernel(xw1, srcp, dstp, nrm)
    _, xw2 = _tc_layer(acc1, xw1, dinv, b1p, w2p)
    acc2 = _agg_kernel(xw2, srcp, dstp, nrm)
    _, xw3 = _tc_layer(acc2, xw2, dinv, b2p, w3p)
    acc3 = _agg_kernel(xw3, srcp, dstp, nrm)
    h3, _ = _tc_layer(acc3, xw3, dinv, b3p, w3p, last=True)

    pmax, psum = _pool_kernel(h3, batchp)
    return _tc_head(pmax, psum, wmax, wmean, bl2)
